# baseline (speedup 1.0000x reference)
import functools

import jax
import jax.numpy as jnp
from jax import lax
from jax.experimental import pallas as pl
from jax.experimental.pallas import tpu as pltpu
from jax.experimental.pallas import tpu_sc as plsc

_N = 8192
_K = 8192
_D = 32
_TILE = 1024
_KB = 1024

_NC = 2
_NS = 16
_NW = _NC * _NS
_BPW = _N // _NW
_DP = 128


def _wn_kernel(w_ref, wn_ref):
    w = w_ref[...]
    wn_ref[...] = jnp.sum(w * w, axis=1)[None, :]


def _nn_kernel(z_ref, w_ref, wn_ref, idx_ref):
    zt = z_ref[...]
    z2 = zt + zt
    wn = wn_ref[...]
    zn = jnp.sum(zt * zt, axis=1, keepdims=True)

    def sim_blk(kb):
        wblk = w_ref[kb * _KB:(kb + 1) * _KB, :]
        dot2 = lax.dot_general(z2, wblk, (((1,), (1,)), ((), ())),
                               preferred_element_type=jnp.float32)
        return dot2 - (zn + wn[:, kb * _KB:(kb + 1) * _KB])

    run_v = sim_blk(0)
    run_b = jnp.zeros((_TILE, _KB), jnp.int32)
    for kb in range(1, _K // _KB):
        sblk = sim_blk(kb)
        upd = sblk > run_v
        run_v = jnp.maximum(run_v, sblk)
        run_b = jnp.where(upd, kb, run_b)

    kiota = lax.broadcasted_iota(jnp.int32, (_TILE, _KB), 1)
    run_i = run_b * _KB + kiota
    m = jnp.max(run_v, axis=1, keepdims=True)
    idx = jnp.min(jnp.where(run_v == m, run_i, _K), axis=1)
    idx_ref[...] = idx.reshape(1, 1, _TILE)


def _gather_body(table_hbm, idx_hbm, out_hbm, idx_v, rows_v, sem):
    wid = lax.axis_index("s") * _NC + lax.axis_index("c")
    base = wid * _BPW
    pltpu.sync_copy(idx_hbm.at[pl.ds(base, _BPW)], idx_v)
    pltpu.async_copy(table_hbm.at[idx_v], rows_v, sem).wait()
    pltpu.sync_copy(rows_v, out_hbm.at[pl.ds(base, _BPW)])


@functools.lru_cache(maxsize=1)
def _make_sc_gather():
    return functools.partial(
        pl.kernel,
        mesh=plsc.VectorSubcoreMesh(core_axis_name="c", subcore_axis_name="s"),
        out_type=jax.ShapeDtypeStruct((_N, _DP), jnp.float32),
        scratch_types=[
            pltpu.VMEM((_BPW,), jnp.int32),
            pltpu.VMEM((_BPW, _DP), jnp.float32),
            pltpu.SemaphoreType.DMA,
        ],
    )(_gather_body)


@jax.jit
def kernel(z, W):
    shape = z.shape
    zf = z.reshape(_N, _D)
    wn = pl.pallas_call(
        _wn_kernel,
        in_specs=[pl.BlockSpec((_K, _D), lambda: (0, 0))],
        out_specs=pl.BlockSpec((1, _K), lambda: (0, 0)),
        out_shape=jax.ShapeDtypeStruct((1, _K), jnp.float32),
    )(W)
    idx = pl.pallas_call(
        _nn_kernel,
        grid=(_N // _TILE,),
        in_specs=[
            pl.BlockSpec((_TILE, _D), lambda i: (i, 0)),
            pl.BlockSpec((_K, _D), lambda i: (0, 0)),
            pl.BlockSpec((1, _K), lambda i: (0, 0)),
        ],
        out_specs=pl.BlockSpec((1, 1, _TILE), lambda i: (i, 0, 0)),
        out_shape=jax.ShapeDtypeStruct((_N // _TILE, 1, _TILE), jnp.int32),
        compiler_params=pltpu.CompilerParams(
            dimension_semantics=("parallel",),
        ),
    )(zf, W, wn)
    w_pad = jnp.pad(W, ((0, 0), (0, _DP - _D)))
    zq = _make_sc_gather()(w_pad, idx.reshape(_N))
    indices = idx.reshape(*shape[:-1], 1)
    z_q = zq[:, :_D].reshape(shape)
    loss = jnp.zeros((1,), dtype=z.dtype)
    return (z_q, indices, loss)


# speedup vs baseline: 1.0233x; 1.0233x over previous
import functools

import jax
import jax.numpy as jnp
from jax import lax
from jax.experimental import pallas as pl
from jax.experimental.pallas import tpu as pltpu
from jax.experimental.pallas import tpu_sc as plsc

_N = 8192
_K = 8192
_D = 32
_TILE = 1024
_KB = 1024

_NC = 2
_NS = 16
_NW = _NC * _NS
_BPW = _N // _NW
_DP = 128


def _prep_kernel(w_ref, wn_ref, wpad_ref):
    w = w_ref[...]
    wn_ref[...] = jnp.sum(w * w, axis=1)[None, :]
    wpad_ref[:, 0:_D] = w


def _nn_kernel(z_ref, w_ref, wn_ref, idx_ref):
    zt = z_ref[...]
    z2 = zt + zt
    wn = wn_ref[...]
    zn = jnp.sum(zt * zt, axis=1, keepdims=True)

    dot2 = lax.dot_general(z2, w_ref[...], (((1,), (1,)), ((), ())),
                           preferred_element_type=jnp.float32)

    def sim_blk(kb):
        return dot2[:, kb * _KB:(kb + 1) * _KB] - (zn + wn[:, kb * _KB:(kb + 1) * _KB])

    run_v = sim_blk(0)
    run_b = jnp.zeros((_TILE, _KB), jnp.int32)
    for kb in range(1, _K // _KB):
        sblk = sim_blk(kb)
        upd = sblk > run_v
        run_v = jnp.maximum(run_v, sblk)
        run_b = jnp.where(upd, kb, run_b)

    kiota = lax.broadcasted_iota(jnp.int32, (_TILE, _KB), 1)
    run_i = run_b * _KB + kiota
    m = jnp.max(run_v, axis=1, keepdims=True)
    idx = jnp.min(jnp.where(run_v == m, run_i, _K), axis=1)
    idx_ref[...] = idx.reshape(1, 1, _TILE)


def _gather_body(table_hbm, idx_hbm, out_hbm, idx_v, rows_v, sem):
    wid = lax.axis_index("s") * _NC + lax.axis_index("c")
    base = wid * _BPW
    pltpu.sync_copy(idx_hbm.at[pl.ds(base, _BPW)], idx_v)
    pltpu.async_copy(table_hbm.at[idx_v], rows_v, sem).wait()
    pltpu.sync_copy(rows_v, out_hbm.at[pl.ds(base, _BPW)])


@functools.lru_cache(maxsize=1)
def _make_sc_gather():
    return functools.partial(
        pl.kernel,
        mesh=plsc.VectorSubcoreMesh(core_axis_name="c", subcore_axis_name="s"),
        out_type=jax.ShapeDtypeStruct((_N, _DP), jnp.float32),
        scratch_types=[
            pltpu.VMEM((_BPW,), jnp.int32),
            pltpu.VMEM((_BPW, _DP), jnp.float32),
            pltpu.SemaphoreType.DMA,
        ],
    )(_gather_body)


@jax.jit
def kernel(z, W):
    shape = z.shape
    zf = z.reshape(_N, _D)
    wn, w_pad = pl.pallas_call(
        _prep_kernel,
        in_specs=[pl.BlockSpec((_K, _D), lambda: (0, 0))],
        out_specs=[
            pl.BlockSpec((1, _K), lambda: (0, 0)),
            pl.BlockSpec((_K, _DP), lambda: (0, 0)),
        ],
        out_shape=[
            jax.ShapeDtypeStruct((1, _K), jnp.float32),
            jax.ShapeDtypeStruct((_K, _DP), jnp.float32),
        ],
    )(W)
    idx = pl.pallas_call(
        _nn_kernel,
        grid=(_N // _TILE,),
        in_specs=[
            pl.BlockSpec((_TILE, _D), lambda i: (i, 0)),
            pl.BlockSpec((_K, _D), lambda i: (0, 0)),
            pl.BlockSpec((1, _K), lambda i: (0, 0)),
        ],
        out_specs=pl.BlockSpec((1, 1, _TILE), lambda i: (i, 0, 0)),
        out_shape=jax.ShapeDtypeStruct((_N // _TILE, 1, _TILE), jnp.int32),
        compiler_params=pltpu.CompilerParams(
            dimension_semantics=("parallel",),
        ),
    )(zf, W, wn)
    zq = _make_sc_gather()(w_pad, idx.reshape(_N))
    indices = idx.reshape(*shape[:-1], 1)
    z_q = zq[:, :_D].reshape(shape)
    loss = jnp.zeros((1,), dtype=z.dtype)
    return (z_q, indices, loss)


# speedup vs baseline: 1.0282x; 1.0048x over previous
import functools

import jax
import jax.numpy as jnp
from jax import lax
from jax.experimental import pallas as pl
from jax.experimental.pallas import tpu as pltpu
from jax.experimental.pallas import tpu_sc as plsc

_N = 8192
_K = 8192
_D = 32
_TILE = 1024
_KB = 1024

_NC = 2
_NS = 16
_NW = _NC * _NS
_BPW = _N // _NW
_DP = 128


def _prep_kernel(z_ref, w_ref, zn_ref, wn_ref, w2_ref, wpad_ref):
    zf = z_ref[...]
    w = w_ref[...]
    zn_col = jnp.sum(zf * zf, axis=1, keepdims=True)
    zn_ref[...] = zn_col.reshape(1, _N)
    wn_ref[...] = jnp.sum(w * w, axis=1, keepdims=True)
    w2_ref[...] = w + w
    wpad_ref[:, 0:_D] = w


def _nn_kernel(zt_ref, w2_ref, wn_ref, zn_ref, idx_ref):
    zt = zt_ref[...]
    zn = zn_ref[...]
    dot2 = lax.dot_general(w2_ref[...], zt, (((1,), (0,)), ((), ())),
                           preferred_element_type=jnp.float32)

    def sim_blk(kb):
        return (dot2[kb * _KB:(kb + 1) * _KB, :]
                - (zn + wn_ref[kb * _KB:(kb + 1) * _KB, :]))

    run_v = sim_blk(0)
    run_b = jnp.zeros((_KB, _TILE), jnp.int32)
    for kb in range(1, _K // _KB):
        sblk = sim_blk(kb)
        upd = sblk > run_v
        run_v = jnp.maximum(run_v, sblk)
        run_b = jnp.where(upd, kb, run_b)

    riota = lax.broadcasted_iota(jnp.int32, (_KB, _TILE), 0)
    run_i = run_b * _KB + riota
    m = jnp.max(run_v, axis=0, keepdims=True)
    idx = jnp.min(jnp.where(run_v == m, run_i, _K), axis=0)
    idx_ref[...] = idx.reshape(1, 1, _TILE)


def _gather_body(table_hbm, idx_hbm, out_hbm, idx_v, rows_v, sem):
    wid = lax.axis_index("s") * _NC + lax.axis_index("c")
    base = wid * _BPW
    pltpu.sync_copy(idx_hbm.at[pl.ds(base, _BPW)], idx_v)
    pltpu.async_copy(table_hbm.at[idx_v], rows_v, sem).wait()
    pltpu.sync_copy(rows_v, out_hbm.at[pl.ds(base, _BPW)])


@functools.lru_cache(maxsize=1)
def _make_sc_gather():
    return functools.partial(
        pl.kernel,
        mesh=plsc.VectorSubcoreMesh(core_axis_name="c", subcore_axis_name="s"),
        out_type=jax.ShapeDtypeStruct((_N, _DP), jnp.float32),
        scratch_types=[
            pltpu.VMEM((_BPW,), jnp.int32),
            pltpu.VMEM((_BPW, _DP), jnp.float32),
            pltpu.SemaphoreType.DMA,
        ],
    )(_gather_body)


@jax.jit
def kernel(z, W):
    shape = z.shape
    zf = z.reshape(_N, _D)
    zn, wn, w2, w_pad = pl.pallas_call(
        _prep_kernel,
        in_specs=[
            pl.BlockSpec((_N, _D), lambda: (0, 0)),
            pl.BlockSpec((_K, _D), lambda: (0, 0)),
        ],
        out_specs=[
            pl.BlockSpec((1, _N), lambda: (0, 0)),
            pl.BlockSpec((_K, 1), lambda: (0, 0)),
            pl.BlockSpec((_K, _D), lambda: (0, 0)),
            pl.BlockSpec((_K, _DP), lambda: (0, 0)),
        ],
        out_shape=[
            jax.ShapeDtypeStruct((1, _N), jnp.float32),
            jax.ShapeDtypeStruct((_K, 1), jnp.float32),
            jax.ShapeDtypeStruct((_K, _D), jnp.float32),
            jax.ShapeDtypeStruct((_K, _DP), jnp.float32),
        ],
    )(zf, W)
    zt = zf.T
    idx = pl.pallas_call(
        _nn_kernel,
        grid=(_N // _TILE,),
        in_specs=[
            pl.BlockSpec((_D, _TILE), lambda i: (0, i)),
            pl.BlockSpec((_K, _D), lambda i: (0, 0)),
            pl.BlockSpec((_K, 1), lambda i: (0, 0)),
            pl.BlockSpec((1, _TILE), lambda i: (0, i)),
        ],
        out_specs=pl.BlockSpec((1, 1, _TILE), lambda i: (i, 0, 0)),
        out_shape=jax.ShapeDtypeStruct((_N // _TILE, 1, _TILE), jnp.int32),
        compiler_params=pltpu.CompilerParams(
            dimension_semantics=("parallel",),
        ),
    )(zt, w2, wn, zn)
    zq = _make_sc_gather()(w_pad, idx.reshape(_N))
    indices = idx.reshape(*shape[:-1], 1)
    z_q = zq[:, :_D].reshape(shape)
    loss = jnp.zeros((1,), dtype=z.dtype)
    return (z_q, indices, loss)
